# Initial kernel scaffold; baseline (speedup 1.0000x reference)
import functools

import jax
import jax.numpy as jnp
from jax import lax
from jax.experimental import pallas as pl
from jax.experimental.pallas import tpu as pltpu
from jax.experimental.pallas import tpu_sc as plsc

M = 100000
D = 384
HID = 384
K = 8

BLK = 4000
N_BLK = M // BLK

LANES = 16
N_CORES = 2
N_SUB = 16
N_TILES = N_CORES * N_SUB
PAD_M = 102400
CHUNK = PAD_M // N_TILES
NEG = -3.0e38

_PREC = lax.Precision.HIGHEST


def _sim_body(q_ref, sw_ref, mem_ref, coords_ref, out_ref):
    q = q_ref[...]
    qn = q / jnp.maximum(jnp.sqrt(jnp.sum(q * q)), 1e-8)
    mem = mem_ref[...]
    sim_raw = lax.dot_general(
        mem, qn, (((1,), (1,)), ((), ())),
        preferred_element_type=jnp.float32, precision=_PREC)
    ones = jnp.ones((1, D), jnp.float32)
    normsq = lax.dot_general(
        mem * mem, ones, (((1,), (1,)), ((), ())),
        preferred_element_type=jnp.float32, precision=_PREC)
    sim = sim_raw / jnp.maximum(jnp.sqrt(normsq), 1e-8)

    sw = sw_ref[...]
    center = jnp.mean(sw, axis=0, keepdims=True)
    diff = coords_ref[...] - center
    dist = jnp.sqrt(jnp.sum(diff * diff, axis=1, keepdims=True))
    out_ref[...] = sim + 1.0 / (1.0 + dist)


def _total_sim(query_vector, spatial_weights, memory_vectors, coords):
    return pl.pallas_call(
        _sim_body,
        grid=(N_BLK,),
        in_specs=[
            pl.BlockSpec((1, D), lambda i: (0, 0)),
            pl.BlockSpec((D, 2), lambda i: (0, 0)),
            pl.BlockSpec((BLK, D), lambda i: (i, 0)),
            pl.BlockSpec((BLK, 2), lambda i: (i, 0)),
        ],
        out_specs=pl.BlockSpec((BLK, 1), lambda i: (i, 0)),
        out_shape=jax.ShapeDtypeStruct((M, 1), jnp.float32),
    )(query_vector, spatial_weights, memory_vectors, coords)


def _merge_desc(av, ai, bv_asc, bi_asc):
    ge = av >= bv_asc
    mv = jnp.where(ge, av, bv_asc)
    mi = jnp.where(ge, ai, bi_asc)
    return plsc.sort_key_val(mv, mi, descending=True)


def _topk_body(sim_hbm, mem_hbm, vals_out, rows_out,
               chunk_v, mv_v, mi_v, sh_vals, sh_idx, tv_v, ti_v, rows_v, sem):
    c = lax.axis_index("c")
    s = lax.axis_index("s")
    base = (c * N_SUB + s) * CHUNK
    pltpu.sync_copy(sim_hbm.at[pl.ds(base, CHUNK)], chunk_v)

    lane = lax.iota(jnp.int32, 16)

    def step(i, carry):
        av, ai = carry
        v = chunk_v[pl.ds(i * 16, 16)]

        def do_merge(_):
            gidx = base + i * 16 + lane
            bv, bi = plsc.sort_key_val(v, gidx, descending=False)
            return _merge_desc(av, ai, bv, bi)

        return lax.cond(jnp.max(v) > jnp.min(av), do_merge,
                        lambda _: (av, ai), None)

    init = (jnp.full((16,), NEG, jnp.float32), jnp.zeros((16,), jnp.int32))
    av, ai = lax.fori_loop(0, CHUNK // 16, step, init)

    mv_v[...] = av
    mi_v[...] = ai
    pltpu.sync_copy(mv_v, sh_vals.at[s])
    pltpu.sync_copy(mi_v, sh_idx.at[s])
    plsc.subcore_barrier()

    @pl.when(s == 0)
    def _():
        def merge_tile(t, carry):
            fv, fi = carry
            pltpu.sync_copy(sh_vals.at[t], tv_v)
            pltpu.sync_copy(sh_idx.at[t], ti_v)
            bv, bi = plsc.sort_key_val(tv_v[...], ti_v[...], descending=False)
            return _merge_desc(fv, fi, bv, bi)

        fv, fi = lax.fori_loop(1, N_SUB, merge_tile, (av, ai))
        mv_v[...] = fv
        mi_v[...] = fi
        pltpu.async_copy(mem_hbm.at[mi_v], rows_v, sem).wait()
        pltpu.sync_copy(mv_v, vals_out.at[c])
        pltpu.sync_copy(rows_v, rows_out.at[c])


_topk_sc = functools.partial(
    pl.kernel,
    mesh=plsc.VectorSubcoreMesh(core_axis_name="c", subcore_axis_name="s"),
    out_type=[
        jax.ShapeDtypeStruct((N_CORES, LANES), jnp.float32),
        jax.ShapeDtypeStruct((N_CORES, LANES, D), jnp.float32),
    ],
    scratch_types=[
        pltpu.VMEM((CHUNK,), jnp.float32),
        pltpu.VMEM((LANES,), jnp.float32),
        pltpu.VMEM((LANES,), jnp.int32),
        pltpu.VMEM_SHARED((N_SUB, LANES), jnp.float32),
        pltpu.VMEM_SHARED((N_SUB, LANES), jnp.int32),
        pltpu.VMEM((LANES,), jnp.float32),
        pltpu.VMEM((LANES,), jnp.int32),
        pltpu.VMEM((LANES, D), jnp.float32),
        pltpu.SemaphoreType.DMA,
    ],
)(_topk_body)


def _rnn_body(q_ref, vr_ref, vc_ref, rows_ref, wih_ref, bih_ref,
              whh_ref, bhh_ref, wg_ref, bg_ref, out_ref):
    q = q_ref[...]
    vr = vr_ref[...]
    vc = vc_ref[...]
    gt = (vc > vr).astype(jnp.float32)
    rank = jnp.sum(gt, axis=0, keepdims=True)
    sel = (lax.broadcasted_iota(jnp.float32, (K, N_TILES), 0)
           == rank).astype(jnp.float32)
    retr = lax.dot_general(
        sel, rows_ref[...], (((1,), (0,)), ((), ())),
        preferred_element_type=jnp.float32, precision=_PREC)

    x = jnp.concatenate([q, retr], axis=0)
    xp = lax.dot_general(
        x, wih_ref[...], (((1,), (0,)), ((), ())),
        preferred_element_type=jnp.float32, precision=_PREC) + bih_ref[...]

    whh = whh_ref[...]
    bhh = bhh_ref[...]
    h = jnp.zeros((1, HID), jnp.float32)
    for t in range(K + 1):
        h = jnp.tanh(xp[t:t + 1, :] + lax.dot_general(
            h, whh, (((1,), (0,)), ((), ())),
            preferred_element_type=jnp.float32, precision=_PREC) + bhh)

    gate = jax.nn.sigmoid(lax.dot_general(
        q, wg_ref[...], (((1,), (0,)), ((), ())),
        preferred_element_type=jnp.float32, precision=_PREC) + bg_ref[...])
    out_ref[...] = gate * h + (1.0 - gate) * xp[0:1, :]


def _contextualize(query_vector, vals, rows, W_ih, b_ih, W_hh, b_hh,
                   W_gate, b_gate):
    full = lambda s: pl.BlockSpec(s, lambda: tuple(0 for _ in s))
    return pl.pallas_call(
        _rnn_body,
        in_specs=[
            full((1, D)), full((1, N_TILES)), full((N_TILES, 1)),
            full((N_TILES, D)), full((D, HID)), full((1, HID)),
            full((HID, HID)), full((1, HID)), full((D, HID)), full((1, HID)),
        ],
        out_specs=full((1, HID)),
        out_shape=jax.ShapeDtypeStruct((1, HID), jnp.float32),
    )(query_vector, vals.reshape(1, N_TILES), vals.reshape(N_TILES, 1),
      rows.reshape(N_TILES, D), W_ih, b_ih.reshape(1, HID), W_hh,
      b_hh.reshape(1, HID), W_gate, b_gate.reshape(1, HID))


def kernel(query_vector, memory_vectors, coords, spatial_weights,
           W_ih, b_ih, W_hh, b_hh, W_gate, b_gate):
    sim = _total_sim(query_vector, spatial_weights, memory_vectors, coords)
    sim_flat = jnp.concatenate(
        [sim.reshape(M), jnp.full((PAD_M - M,), NEG, jnp.float32)])
    vals, rows = _topk_sc(sim_flat, memory_vectors)
    return _contextualize(query_vector, vals, rows, W_ih, b_ih, W_hh, b_hh,
                          W_gate, b_gate)


# speedup vs baseline: 1.4748x; 1.4748x over previous
import dataclasses
import functools

import jax
import jax.numpy as jnp
from jax import lax
from jax.experimental import pallas as pl
from jax.experimental.pallas import tpu as pltpu
from jax.experimental.pallas import tpu_sc as plsc

M = 100000
D = 384
HID = 384
K = 8

BLK = 4000
N_BLK = M // BLK

LANES = 16
N_CORES = 2
N_SUB = 16
N_TILES = N_CORES * N_SUB
PAD_M = 102400
CHUNK = PAD_M // N_TILES
NEG = -3.0e38

_PREC = lax.Precision.HIGHEST


def _sim_body(q_ref, sw_ref, mem_ref, coords_ref, out_ref):
    q = q_ref[...]
    qn = q / jnp.maximum(jnp.sqrt(jnp.sum(q * q)), 1e-8)
    mem = mem_ref[...]
    sim_raw = lax.dot_general(
        mem, qn, (((1,), (1,)), ((), ())),
        preferred_element_type=jnp.float32, precision=_PREC)
    ones = jnp.ones((1, D), jnp.float32)
    normsq = lax.dot_general(
        mem * mem, ones, (((1,), (1,)), ((), ())),
        preferred_element_type=jnp.float32, precision=_PREC)
    sim = sim_raw / jnp.maximum(jnp.sqrt(normsq), 1e-8)

    sw = sw_ref[...]
    center = jnp.mean(sw, axis=0, keepdims=True)
    diff = coords_ref[...] - center
    dist = jnp.sqrt(jnp.sum(diff * diff, axis=1, keepdims=True))
    out_ref[...] = sim + 1.0 / (1.0 + dist)


def _total_sim(query_vector, spatial_weights, memory_vectors, coords):
    return pl.pallas_call(
        _sim_body,
        grid=(N_BLK,),
        in_specs=[
            pl.BlockSpec((1, D), lambda i: (0, 0)),
            pl.BlockSpec((D, 2), lambda i: (0, 0)),
            pl.BlockSpec((BLK, D), lambda i: (i, 0)),
            pl.BlockSpec((BLK, 2), lambda i: (i, 0)),
        ],
        out_specs=pl.BlockSpec((BLK, 1), lambda i: (i, 0)),
        out_shape=jax.ShapeDtypeStruct((M, 1), jnp.float32),
    )(query_vector, spatial_weights, memory_vectors, coords)


def _merge_desc(av, ai, bv_asc, bi_asc):
    ge = av >= bv_asc
    mv = jnp.where(ge, av, bv_asc)
    mi = jnp.where(ge, ai, bi_asc)
    sv, si = plsc.sort_key_val(mv, mi, descending=True)
    return sv, si


def _topk_body(sim_hbm, mem_hbm, vals_out, rows_out, pv_hbm, pi_hbm,
               chunk_v, mv_v, mi_v, tv_v, ti_v, rows_v, sem):
    c = lax.axis_index("c")
    s = lax.axis_index("s")
    w = c * N_SUB + s
    base = w * CHUNK
    pltpu.sync_copy(sim_hbm.at[pl.ds(base, CHUNK)], chunk_v)

    lane = lax.iota(jnp.int32, 16)

    def step(i, carry):
        av, ai = carry
        v = chunk_v[pl.ds(i * 16, 16)]
        gidx = base + i * 16 + lane
        bv, bi = plsc.sort_key_val(v, gidx, descending=False)
        return _merge_desc(av, ai, bv, bi)

    init = (jnp.full((16,), NEG, jnp.float32), jnp.zeros((16,), jnp.int32))
    av, ai = lax.fori_loop(0, CHUNK // 16, step, init)

    mv_v[...] = av
    mi_v[...] = ai
    pltpu.sync_copy(mv_v, pv_hbm.at[w])
    pltpu.sync_copy(mi_v, pi_hbm.at[w])
    plsc.subcore_barrier()

    @pl.when(s == 0)
    def _():
        def merge_tile(t, carry):
            fv, fi = carry
            pltpu.sync_copy(pv_hbm.at[c * N_SUB + t], tv_v)
            pltpu.sync_copy(pi_hbm.at[c * N_SUB + t], ti_v)
            bv, bi = plsc.sort_key_val(tv_v[...], ti_v[...], descending=False)
            return _merge_desc(fv, fi, bv, bi)

        fv, fi = lax.fori_loop(1, N_SUB, merge_tile, (av, ai))
        mv_v[...] = fv
        mi_v[...] = fi
        pltpu.async_copy(mem_hbm.at[mi_v], rows_v, sem).wait()
        pltpu.sync_copy(mv_v, vals_out.at[c])
        pltpu.sync_copy(rows_v, rows_out.at[c])


@functools.cache
def _make_topk_sc():
    cp = pltpu.CompilerParams()
    if "needs_layout_passes" in pltpu.CompilerParams.__dataclass_fields__:
        cp = dataclasses.replace(cp, needs_layout_passes=False)
    return functools.partial(
        pl.kernel,
        compiler_params=cp,
        mesh=plsc.VectorSubcoreMesh(core_axis_name="c", subcore_axis_name="s"),
        out_type=[
            jax.ShapeDtypeStruct((N_CORES, LANES), jnp.float32),
            jax.ShapeDtypeStruct((N_CORES, LANES, D), jnp.float32),
            jax.ShapeDtypeStruct((N_TILES, LANES), jnp.float32),
            jax.ShapeDtypeStruct((N_TILES, LANES), jnp.int32),
        ],
        scratch_types=[
            pltpu.VMEM((CHUNK,), jnp.float32),
            pltpu.VMEM((LANES,), jnp.float32),
            pltpu.VMEM((LANES,), jnp.int32),
            pltpu.VMEM((LANES,), jnp.float32),
            pltpu.VMEM((LANES,), jnp.int32),
            pltpu.VMEM((LANES, D), jnp.float32),
            pltpu.SemaphoreType.DMA,
        ],
    )(_topk_body)


def _rnn_body(q_ref, vr_ref, vc_ref, rows_ref, wih_ref, bih_ref,
              whh_ref, bhh_ref, wg_ref, bg_ref, out_ref):
    q = q_ref[...]
    vr = vr_ref[...]
    vc = vc_ref[...]
    gt = (vc > vr).astype(jnp.int32)
    rank = jnp.sum(gt, axis=0, keepdims=True)
    sel = (lax.broadcasted_iota(jnp.int32, (K, N_TILES), 0)
           == rank).astype(jnp.float32)
    retr = lax.dot_general(
        sel, rows_ref[...], (((1,), (0,)), ((), ())),
        preferred_element_type=jnp.float32, precision=_PREC)

    x = jnp.concatenate([q, retr], axis=0)
    xp = lax.dot_general(
        x, wih_ref[...], (((1,), (0,)), ((), ())),
        preferred_element_type=jnp.float32, precision=_PREC) + bih_ref[...]

    whh = whh_ref[...]
    bhh = bhh_ref[...]
    h = jnp.zeros((1, HID), jnp.float32)
    for t in range(K + 1):
        h = jnp.tanh(xp[t:t + 1, :] + lax.dot_general(
            h, whh, (((1,), (0,)), ((), ())),
            preferred_element_type=jnp.float32, precision=_PREC) + bhh)

    gate = jax.nn.sigmoid(lax.dot_general(
        q, wg_ref[...], (((1,), (0,)), ((), ())),
        preferred_element_type=jnp.float32, precision=_PREC) + bg_ref[...])
    out_ref[...] = gate * h + (1.0 - gate) * xp[0:1, :]


def _contextualize(query_vector, vals, rows, W_ih, b_ih, W_hh, b_hh,
                   W_gate, b_gate):
    full = lambda s: pl.BlockSpec(s, lambda: tuple(0 for _ in s))
    return pl.pallas_call(
        _rnn_body,
        in_specs=[
            full((1, D)), full((1, N_TILES)), full((N_TILES, 1)),
            full((N_TILES, D)), full((D, HID)), full((1, HID)),
            full((HID, HID)), full((1, HID)), full((D, HID)), full((1, HID)),
        ],
        out_specs=full((1, HID)),
        out_shape=jax.ShapeDtypeStruct((1, HID), jnp.float32),
    )(query_vector, vals.reshape(1, N_TILES), vals.reshape(N_TILES, 1),
      rows.reshape(N_TILES, D), W_ih, b_ih.reshape(1, HID), W_hh,
      b_hh.reshape(1, HID), W_gate, b_gate.reshape(1, HID))


def kernel(query_vector, memory_vectors, coords, spatial_weights,
           W_ih, b_ih, W_hh, b_hh, W_gate, b_gate):
    sim = _total_sim(query_vector, spatial_weights, memory_vectors, coords)
    sim_flat = jnp.concatenate(
        [sim.reshape(M), jnp.full((PAD_M - M,), NEG, jnp.float32)])
    vals, rows, _, _ = _make_topk_sc()(sim_flat, memory_vectors)
    return _contextualize(query_vector, vals, rows, W_ih, b_ih, W_hh, b_hh,
                          W_gate, b_gate)


# speedup vs baseline: 2.1054x; 1.4276x over previous
import dataclasses
import functools

import jax
import jax.numpy as jnp
from jax import lax
from jax.experimental import pallas as pl
from jax.experimental.pallas import tpu as pltpu
from jax.experimental.pallas import tpu_sc as plsc

M = 100000
D = 384
HID = 384
K = 8

BLK = 4096
N_BLK = 25

LANES = 16
N_CORES = 2
N_SUB = 16
N_TILES = N_CORES * N_SUB
PAD_M = 102400
CHUNK = PAD_M // N_TILES
NEG = -3.0e38

_PREC = lax.Precision.HIGHEST


def _sim_body(q_ref, sw_ref, mem_ref, ct_ref, sim_ref, act_ref):
    q = q_ref[...]
    qn = q / jnp.maximum(jnp.sqrt(jnp.sum(q * q)), 1e-8)
    mem = mem_ref[...]
    sim_raw = lax.dot_general(
        mem, qn, (((1,), (1,)), ((), ())),
        preferred_element_type=jnp.float32, precision=_PREC)
    ones = jnp.ones((1, D), jnp.float32)
    normsq = lax.dot_general(
        mem * mem, ones, (((1,), (1,)), ((), ())),
        preferred_element_type=jnp.float32, precision=_PREC)
    sim_ref[...] = sim_raw * lax.rsqrt(jnp.maximum(normsq, 1e-16))

    sw = sw_ref[...]
    cx = jnp.mean(sw[:, 0])
    cy = jnp.mean(sw[:, 1])
    ct = ct_ref[...]
    dx = ct[0] - cx
    dy = ct[1] - cy
    act_ref[...] = 1.0 / (1.0 + jnp.sqrt(dx * dx + dy * dy))


def _total_sim(query_vector, spatial_weights, memory_vectors, coords):
    coords_t = jnp.pad(coords, ((0, PAD_M - M), (0, 0))).T.reshape(
        2, PAD_M // 128, 128)
    return pl.pallas_call(
        _sim_body,
        grid=(N_BLK,),
        in_specs=[
            pl.BlockSpec((1, D), lambda i: (0, 0)),
            pl.BlockSpec((D, 2), lambda i: (0, 0)),
            pl.BlockSpec((BLK, D), lambda i: (i, 0)),
            pl.BlockSpec((2, BLK // 128, 128), lambda i: (0, i, 0)),
        ],
        out_specs=[
            pl.BlockSpec((BLK, 1), lambda i: (i, 0)),
            pl.BlockSpec((BLK // 128, 128), lambda i: (i, 0)),
        ],
        out_shape=[
            jax.ShapeDtypeStruct((PAD_M, 1), jnp.float32),
            jax.ShapeDtypeStruct((PAD_M // 128, 128), jnp.float32),
        ],
    )(query_vector, spatial_weights, memory_vectors, coords_t)


def _merge_desc(av, ai, bv_asc, bi_asc):
    ge = av >= bv_asc
    mv = jnp.where(ge, av, bv_asc)
    mi = jnp.where(ge, ai, bi_asc)
    sv, si = plsc.sort_key_val(mv, mi, descending=True)
    return sv, si


def _topk_body(sim_hbm, act_hbm, mem_hbm, vals_out, rows_out, pv_hbm, pi_hbm,
               chunk_v, actc_v, mv_v, mi_v, tv_v, ti_v, rows_v, sem):
    c = lax.axis_index("c")
    s = lax.axis_index("s")
    w = c * N_SUB + s
    base = w * CHUNK
    pltpu.sync_copy(sim_hbm.at[pl.ds(base, CHUNK)], chunk_v)
    pltpu.sync_copy(act_hbm.at[pl.ds(base, CHUNK)], actc_v)

    lane = lax.iota(jnp.int32, 16)

    def step(i, carry):
        av, ai = carry
        gidx = base + i * 16 + lane
        v = jnp.where(gidx < M,
                      chunk_v[pl.ds(i * 16, 16)] + actc_v[pl.ds(i * 16, 16)],
                      NEG)
        bv, bi = plsc.sort_key_val(v, gidx, descending=False)
        return _merge_desc(av, ai, bv, bi)

    init = (jnp.full((16,), NEG, jnp.float32), jnp.zeros((16,), jnp.int32))
    av, ai = lax.fori_loop(0, CHUNK // 16, step, init)

    mv_v[...] = av
    mi_v[...] = ai
    pltpu.sync_copy(mv_v, pv_hbm.at[pl.ds(w * LANES, LANES)])
    pltpu.sync_copy(mi_v, pi_hbm.at[pl.ds(w * LANES, LANES)])
    plsc.subcore_barrier()

    @pl.when(s == 0)
    def _():
        pltpu.sync_copy(pv_hbm.at[pl.ds(c * N_SUB * LANES, N_SUB * LANES)],
                        tv_v)
        pltpu.sync_copy(pi_hbm.at[pl.ds(c * N_SUB * LANES, N_SUB * LANES)],
                        ti_v)

        def merge_tile(t, carry):
            fv, fi = carry
            bv, bi = plsc.sort_key_val(tv_v[pl.ds(t * LANES, LANES)],
                                       ti_v[pl.ds(t * LANES, LANES)],
                                       descending=False)
            return _merge_desc(fv, fi, bv, bi)

        fv, fi = lax.fori_loop(1, N_SUB, merge_tile, (av, ai))
        mv_v[...] = fv
        mi_v[...] = fi
        pltpu.async_copy(mem_hbm.at[mi_v], rows_v, sem).wait()
        pltpu.sync_copy(mv_v, vals_out.at[c])
        pltpu.sync_copy(rows_v, rows_out.at[c])


@functools.cache
def _make_topk_sc():
    cp = pltpu.CompilerParams()
    if "needs_layout_passes" in pltpu.CompilerParams.__dataclass_fields__:
        cp = dataclasses.replace(cp, needs_layout_passes=False)
    return functools.partial(
        pl.kernel,
        compiler_params=cp,
        mesh=plsc.VectorSubcoreMesh(core_axis_name="c", subcore_axis_name="s"),
        out_type=[
            jax.ShapeDtypeStruct((N_CORES, LANES), jnp.float32),
            jax.ShapeDtypeStruct((N_CORES, LANES, D), jnp.float32),
            jax.ShapeDtypeStruct((N_TILES * LANES,), jnp.float32),
            jax.ShapeDtypeStruct((N_TILES * LANES,), jnp.int32),
        ],
        scratch_types=[
            pltpu.VMEM((CHUNK,), jnp.float32),
            pltpu.VMEM((CHUNK,), jnp.float32),
            pltpu.VMEM((LANES,), jnp.float32),
            pltpu.VMEM((LANES,), jnp.int32),
            pltpu.VMEM((N_SUB * LANES,), jnp.float32),
            pltpu.VMEM((N_SUB * LANES,), jnp.int32),
            pltpu.VMEM((LANES, D), jnp.float32),
            pltpu.SemaphoreType.DMA,
        ],
    )(_topk_body)


def _rnn_body(q_ref, vr_ref, vc_ref, rows_ref, wih_ref, bih_ref,
              whh_ref, bhh_ref, wg_ref, bg_ref, out_ref):
    q = q_ref[...]
    vr = vr_ref[...]
    vc = vc_ref[...]
    gt = (vc > vr).astype(jnp.int32)
    rank = jnp.sum(gt, axis=0, keepdims=True)
    sel = (lax.broadcasted_iota(jnp.int32, (K, N_TILES), 0)
           == rank).astype(jnp.float32)
    retr = lax.dot_general(
        sel, rows_ref[...], (((1,), (0,)), ((), ())),
        preferred_element_type=jnp.float32, precision=_PREC)

    x = jnp.concatenate([q, retr], axis=0)
    xp = lax.dot_general(
        x, wih_ref[...], (((1,), (0,)), ((), ())),
        preferred_element_type=jnp.float32, precision=_PREC) + bih_ref[...]

    whh = whh_ref[...]
    bhh = bhh_ref[...]
    h = jnp.zeros((1, HID), jnp.float32)
    for t in range(K + 1):
        h = jnp.tanh(xp[t:t + 1, :] + lax.dot_general(
            h, whh, (((1,), (0,)), ((), ())),
            preferred_element_type=jnp.float32, precision=_PREC) + bhh)

    gate = jax.nn.sigmoid(lax.dot_general(
        q, wg_ref[...], (((1,), (0,)), ((), ())),
        preferred_element_type=jnp.float32, precision=_PREC) + bg_ref[...])
    out_ref[...] = gate * h + (1.0 - gate) * xp[0:1, :]


def _contextualize(query_vector, vals, rows, W_ih, b_ih, W_hh, b_hh,
                   W_gate, b_gate):
    full = lambda s: pl.BlockSpec(s, lambda: tuple(0 for _ in s))
    return pl.pallas_call(
        _rnn_body,
        in_specs=[
            full((1, D)), full((1, N_TILES)), full((N_TILES, 1)),
            full((N_TILES, D)), full((D, HID)), full((1, HID)),
            full((HID, HID)), full((1, HID)), full((D, HID)), full((1, HID)),
        ],
        out_specs=full((1, HID)),
        out_shape=jax.ShapeDtypeStruct((1, HID), jnp.float32),
    )(query_vector, vals.reshape(1, N_TILES), vals.reshape(N_TILES, 1),
      rows.reshape(N_TILES, D), W_ih, b_ih.reshape(1, HID), W_hh,
      b_hh.reshape(1, HID), W_gate, b_gate.reshape(1, HID))


def kernel(query_vector, memory_vectors, coords, spatial_weights,
           W_ih, b_ih, W_hh, b_hh, W_gate, b_gate):
    sim, act = _total_sim(query_vector, spatial_weights, memory_vectors,
                          coords)
    vals, rows, _, _ = _make_topk_sc()(sim.reshape(PAD_M), act.reshape(PAD_M),
                                       memory_vectors)
    return _contextualize(query_vector, vals, rows, W_ih, b_ih, W_hh, b_hh,
                          W_gate, b_gate)


# speedup vs baseline: 2.1630x; 1.0274x over previous
import dataclasses
import functools

import jax
import jax.numpy as jnp
from jax import lax
from jax.experimental import pallas as pl
from jax.experimental.pallas import tpu as pltpu
from jax.experimental.pallas import tpu_sc as plsc

M = 100000
D = 384
HID = 384
K = 8

BLK = 4096
N_BLK = 25

LANES = 16
N_CORES = 2
N_SUB = 16
N_TILES = N_CORES * N_SUB
PAD_M = 102400
CHUNK = PAD_M // N_TILES
NEG = -3.0e38

_PREC = lax.Precision.HIGHEST


def _sim_body(q_ref, sw_ref, ct_ref, mem_hbm, sim_hbm, act_ref,
              buf0, buf1, buf2, sb0, sb1, sem0, sem1, sem2, osem0, osem1):
    q = q_ref[...]
    qn = q / jnp.maximum(jnp.sqrt(jnp.sum(q * q)), 1e-8)
    ones = jnp.ones((1, D), jnp.float32)
    sw = sw_ref[...]
    cx = jnp.mean(sw[:, 0])
    cy = jnp.mean(sw[:, 1])
    bufs = (buf0, buf1, buf2)
    sbufs = (sb0, sb1)
    sems = (sem0, sem1, sem2)
    osems = (osem0, osem1)

    def start(i):
        rows = BLK if (i + 1) * BLK <= M else M - i * BLK
        c = pltpu.make_async_copy(mem_hbm.at[pl.ds(i * BLK, rows)],
                                  bufs[i % 3].at[pl.ds(0, rows)],
                                  sems[i % 3])
        c.start()
        return c

    out_copies = [None, None]
    nxt = start(0)
    for i in range(N_BLK):
        cur = nxt
        if i + 1 < N_BLK:
            nxt = start(i + 1)
        cur.wait()
        mem = bufs[i % 3][...]
        sim_raw = lax.dot_general(
            mem, qn, (((1,), (1,)), ((), ())),
            preferred_element_type=jnp.float32, precision=_PREC)
        normsq = lax.dot_general(
            mem * mem, ones, (((1,), (1,)), ((), ())),
            preferred_element_type=jnp.float32, precision=_PREC)
        if out_copies[i % 2] is not None:
            out_copies[i % 2].wait()
        sbufs[i % 2][...] = sim_raw * lax.rsqrt(jnp.maximum(normsq, 1e-16))
        oc = pltpu.make_async_copy(sbufs[i % 2],
                                   sim_hbm.at[pl.ds(i * BLK, BLK)],
                                   osems[i % 2])
        oc.start()
        out_copies[i % 2] = oc
        g = BLK // 128
        ct = ct_ref[:, pl.ds(i * g, g), :]
        dx = ct[0] - cx
        dy = ct[1] - cy
        act_ref[pl.ds(i * g, g), :] = 1.0 / (1.0 + jnp.sqrt(dx * dx + dy * dy))
    out_copies[0].wait()
    out_copies[1].wait()


def _total_sim(query_vector, spatial_weights, memory_vectors, coords):
    coords_t = jnp.pad(coords, ((0, PAD_M - M), (0, 0))).T.reshape(
        2, PAD_M // 128, 128)
    return pl.pallas_call(
        _sim_body,
        in_specs=[
            pl.BlockSpec(memory_space=pltpu.VMEM),
            pl.BlockSpec(memory_space=pltpu.VMEM),
            pl.BlockSpec(memory_space=pltpu.VMEM),
            pl.BlockSpec(memory_space=pltpu.HBM),
        ],
        out_specs=[
            pl.BlockSpec(memory_space=pltpu.HBM),
            pl.BlockSpec(memory_space=pltpu.VMEM),
        ],
        out_shape=[
            jax.ShapeDtypeStruct((PAD_M, 1), jnp.float32),
            jax.ShapeDtypeStruct((PAD_M // 128, 128), jnp.float32),
        ],
        scratch_shapes=[
            pltpu.VMEM((BLK, D), jnp.float32),
            pltpu.VMEM((BLK, D), jnp.float32),
            pltpu.VMEM((BLK, D), jnp.float32),
            pltpu.VMEM((BLK, 1), jnp.float32),
            pltpu.VMEM((BLK, 1), jnp.float32),
            pltpu.SemaphoreType.DMA,
            pltpu.SemaphoreType.DMA,
            pltpu.SemaphoreType.DMA,
            pltpu.SemaphoreType.DMA,
            pltpu.SemaphoreType.DMA,
        ],
    )(query_vector, spatial_weights, coords_t, memory_vectors)


def _merge_desc(av, ai, bv_asc, bi_asc):
    ge = av >= bv_asc
    mv = jnp.where(ge, av, bv_asc)
    mi = jnp.where(ge, ai, bi_asc)
    sv, si = plsc.sort_key_val(mv, mi, descending=True)
    return sv, si


def _topk_body(sim_hbm, act_hbm, mem_hbm, vals_out, rows_out, pv_hbm, pi_hbm,
               chunk_v, actc_v, mv_v, mi_v, tv_v, ti_v, rows_v, sem):
    c = lax.axis_index("c")
    s = lax.axis_index("s")
    w = c * N_SUB + s
    base = w * CHUNK
    pltpu.sync_copy(sim_hbm.at[pl.ds(base, CHUNK)], chunk_v)
    pltpu.sync_copy(act_hbm.at[pl.ds(base, CHUNK)], actc_v)

    lane = lax.iota(jnp.int32, 16)

    def load16(i):
        gidx = base + i * 16 + lane
        v = jnp.where(gidx < M,
                      chunk_v[pl.ds(i * 16, 16)] + actc_v[pl.ds(i * 16, 16)],
                      NEG)
        return v, gidx

    def step(i, carry):
        av0, ai0, av1, ai1 = carry
        v0, g0 = load16(2 * i)
        v1, g1 = load16(2 * i + 1)
        bv0, bi0 = plsc.sort_key_val(v0, g0, descending=False)
        bv1, bi1 = plsc.sort_key_val(v1, g1, descending=False)
        av0, ai0 = _merge_desc(av0, ai0, bv0, bi0)
        av1, ai1 = _merge_desc(av1, ai1, bv1, bi1)
        return av0, ai0, av1, ai1

    neg16 = jnp.full((16,), NEG, jnp.float32)
    zero16 = jnp.zeros((16,), jnp.int32)
    av0, ai0, av1, ai1 = lax.fori_loop(0, CHUNK // 32, step,
                                       (neg16, zero16, neg16, zero16))
    bv1, bi1 = plsc.sort_key_val(av1, ai1, descending=False)
    av, ai = _merge_desc(av0, ai0, bv1, bi1)

    mv_v[...] = av
    mi_v[...] = ai
    pltpu.sync_copy(mv_v, pv_hbm.at[pl.ds(w * LANES, LANES)])
    pltpu.sync_copy(mi_v, pi_hbm.at[pl.ds(w * LANES, LANES)])
    plsc.subcore_barrier()

    @pl.when(s == 0)
    def _():
        pltpu.sync_copy(pv_hbm.at[pl.ds(c * N_SUB * LANES, N_SUB * LANES)],
                        tv_v)
        pltpu.sync_copy(pi_hbm.at[pl.ds(c * N_SUB * LANES, N_SUB * LANES)],
                        ti_v)

        def merge_tile(t, carry):
            fv, fi = carry
            bv, bi = plsc.sort_key_val(tv_v[pl.ds(t * LANES, LANES)],
                                       ti_v[pl.ds(t * LANES, LANES)],
                                       descending=False)
            return _merge_desc(fv, fi, bv, bi)

        fv, fi = lax.fori_loop(1, N_SUB, merge_tile, (av, ai))
        mv_v[...] = fv
        mi_v[...] = fi
        pltpu.async_copy(mem_hbm.at[mi_v], rows_v, sem).wait()
        pltpu.sync_copy(mv_v, vals_out.at[c])
        pltpu.sync_copy(rows_v, rows_out.at[c])


@functools.cache
def _make_topk_sc():
    cp = pltpu.CompilerParams()
    if "needs_layout_passes" in pltpu.CompilerParams.__dataclass_fields__:
        cp = dataclasses.replace(cp, needs_layout_passes=False)
    return functools.partial(
        pl.kernel,
        compiler_params=cp,
        mesh=plsc.VectorSubcoreMesh(core_axis_name="c", subcore_axis_name="s"),
        out_type=[
            jax.ShapeDtypeStruct((N_CORES, LANES), jnp.float32),
            jax.ShapeDtypeStruct((N_CORES, LANES, D), jnp.float32),
            jax.ShapeDtypeStruct((N_TILES * LANES,), jnp.float32),
            jax.ShapeDtypeStruct((N_TILES * LANES,), jnp.int32),
        ],
        scratch_types=[
            pltpu.VMEM((CHUNK,), jnp.float32),
            pltpu.VMEM((CHUNK,), jnp.float32),
            pltpu.VMEM((LANES,), jnp.float32),
            pltpu.VMEM((LANES,), jnp.int32),
            pltpu.VMEM((N_SUB * LANES,), jnp.float32),
            pltpu.VMEM((N_SUB * LANES,), jnp.int32),
            pltpu.VMEM((LANES, D), jnp.float32),
            pltpu.SemaphoreType.DMA,
        ],
    )(_topk_body)


def _rnn_body(q_ref, vr_ref, vc_ref, rows_ref, wih_ref, bih_ref,
              whh_ref, bhh_ref, wg_ref, bg_ref, out_ref):
    q = q_ref[...]
    vr = vr_ref[...]
    vc = vc_ref[...]
    gt = (vc > vr).astype(jnp.int32)
    rank = jnp.sum(gt, axis=0, keepdims=True)
    sel = (lax.broadcasted_iota(jnp.int32, (K, N_TILES), 0)
           == rank).astype(jnp.float32)
    retr = lax.dot_general(
        sel, rows_ref[...], (((1,), (0,)), ((), ())),
        preferred_element_type=jnp.float32, precision=_PREC)

    x = jnp.concatenate([q, retr], axis=0)
    xp = lax.dot_general(
        x, wih_ref[...], (((1,), (0,)), ((), ())),
        preferred_element_type=jnp.float32, precision=_PREC) + bih_ref[...]

    whh = whh_ref[...]
    bhh = bhh_ref[...]
    h = jnp.zeros((1, HID), jnp.float32)
    for t in range(K + 1):
        h = jnp.tanh(xp[t:t + 1, :] + lax.dot_general(
            h, whh, (((1,), (0,)), ((), ())),
            preferred_element_type=jnp.float32, precision=_PREC) + bhh)

    gate = jax.nn.sigmoid(lax.dot_general(
        q, wg_ref[...], (((1,), (0,)), ((), ())),
        preferred_element_type=jnp.float32, precision=_PREC) + bg_ref[...])
    out_ref[...] = gate * h + (1.0 - gate) * xp[0:1, :]


def _contextualize(query_vector, vals, rows, W_ih, b_ih, W_hh, b_hh,
                   W_gate, b_gate):
    full = lambda s: pl.BlockSpec(s, lambda: tuple(0 for _ in s))
    return pl.pallas_call(
        _rnn_body,
        in_specs=[
            full((1, D)), full((1, N_TILES)), full((N_TILES, 1)),
            full((N_TILES, D)), full((D, HID)), full((1, HID)),
            full((HID, HID)), full((1, HID)), full((D, HID)), full((1, HID)),
        ],
        out_specs=full((1, HID)),
        out_shape=jax.ShapeDtypeStruct((1, HID), jnp.float32),
    )(query_vector, vals.reshape(1, N_TILES), vals.reshape(N_TILES, 1),
      rows.reshape(N_TILES, D), W_ih, b_ih.reshape(1, HID), W_hh,
      b_hh.reshape(1, HID), W_gate, b_gate.reshape(1, HID))


def kernel(query_vector, memory_vectors, coords, spatial_weights,
           W_ih, b_ih, W_hh, b_hh, W_gate, b_gate):
    sim, act = _total_sim(query_vector, spatial_weights, memory_vectors,
                          coords)
    vals, rows, _, _ = _make_topk_sc()(sim.reshape(PAD_M), act.reshape(PAD_M),
                                       memory_vectors)
    return _contextualize(query_vector, vals, rows, W_ih, b_ih, W_hh, b_hh,
                          W_gate, b_gate)


# speedup vs baseline: 2.1696x; 1.0031x over previous
import dataclasses
import functools

import jax
import jax.numpy as jnp
from jax import lax
from jax.experimental import pallas as pl
from jax.experimental.pallas import tpu as pltpu
from jax.experimental.pallas import tpu_sc as plsc

M = 100000
D = 384
HID = 384
K = 8

BLK = 5120
N_BLK = 20

LANES = 16
N_CORES = 2
N_SUB = 16
N_TILES = N_CORES * N_SUB
PAD_M = 102400
CHUNK = PAD_M // N_TILES
NEG = -3.0e38

_PREC = lax.Precision.HIGHEST


def _sim_body(q_ref, sw_ref, mem_ref, ct_ref, sim_ref, act_ref):
    q = q_ref[...]
    qn = q / jnp.maximum(jnp.sqrt(jnp.sum(q * q)), 1e-8)
    mem = mem_ref[...]
    sim_raw = lax.dot_general(
        mem, qn, (((1,), (1,)), ((), ())),
        preferred_element_type=jnp.float32, precision=_PREC)
    ones = jnp.ones((1, D), jnp.float32)
    normsq = lax.dot_general(
        mem * mem, ones, (((1,), (1,)), ((), ())),
        preferred_element_type=jnp.float32, precision=_PREC)
    sim_ref[...] = sim_raw * lax.rsqrt(jnp.maximum(normsq, 1e-16))

    sw = sw_ref[...]
    cx = jnp.mean(sw[:, 0])
    cy = jnp.mean(sw[:, 1])
    ct = ct_ref[...]
    dx = ct[0] - cx
    dy = ct[1] - cy
    act_ref[...] = 1.0 / (1.0 + jnp.sqrt(dx * dx + dy * dy))


def _total_sim(query_vector, spatial_weights, memory_vectors, coords):
    coords_t = jnp.pad(coords, ((0, PAD_M - M), (0, 0))).T.reshape(
        2, PAD_M // 128, 128)
    return pl.pallas_call(
        _sim_body,
        grid=(N_BLK,),
        in_specs=[
            pl.BlockSpec((1, D), lambda i: (0, 0)),
            pl.BlockSpec((D, 2), lambda i: (0, 0)),
            pl.BlockSpec((BLK, D), lambda i: (i, 0)),
            pl.BlockSpec((2, BLK // 128, 128), lambda i: (0, i, 0)),
        ],
        out_specs=[
            pl.BlockSpec((BLK, 1), lambda i: (i, 0)),
            pl.BlockSpec((BLK // 128, 128), lambda i: (i, 0)),
        ],
        out_shape=[
            jax.ShapeDtypeStruct((PAD_M, 1), jnp.float32),
            jax.ShapeDtypeStruct((PAD_M // 128, 128), jnp.float32),
        ],
    )(query_vector, spatial_weights, memory_vectors, coords_t)


def _merge_desc(av, ai, bv_asc, bi_asc):
    ge = av >= bv_asc
    mv = jnp.where(ge, av, bv_asc)
    mi = jnp.where(ge, ai, bi_asc)
    sv, si = plsc.sort_key_val(mv, mi, descending=True)
    return sv, si


def _topk_body(sim_hbm, act_hbm, mem_hbm, vals_out, rows_out, pv_hbm, pi_hbm,
               chunk_v, actc_v, mv_v, mi_v, tv_v, ti_v, rows_v, sem):
    c = lax.axis_index("c")
    s = lax.axis_index("s")
    w = c * N_SUB + s
    base = w * CHUNK
    pltpu.sync_copy(sim_hbm.at[pl.ds(base, CHUNK)], chunk_v)
    pltpu.sync_copy(act_hbm.at[pl.ds(base, CHUNK)], actc_v)

    lane = lax.iota(jnp.int32, 16)

    def load16(i):
        gidx = base + i * 16 + lane
        v = jnp.where(gidx < M,
                      chunk_v[pl.ds(i * 16, 16)] + actc_v[pl.ds(i * 16, 16)],
                      NEG)
        return v, gidx

    def step(i, carry):
        av0, ai0, av1, ai1 = carry
        v0, g0 = load16(2 * i)
        v1, g1 = load16(2 * i + 1)
        bv0, bi0 = plsc.sort_key_val(v0, g0, descending=False)
        bv1, bi1 = plsc.sort_key_val(v1, g1, descending=False)
        av0, ai0 = _merge_desc(av0, ai0, bv0, bi0)
        av1, ai1 = _merge_desc(av1, ai1, bv1, bi1)
        return av0, ai0, av1, ai1

    neg16 = jnp.full((16,), NEG, jnp.float32)
    zero16 = jnp.zeros((16,), jnp.int32)
    av0, ai0, av1, ai1 = lax.fori_loop(0, CHUNK // 32, step,
                                       (neg16, zero16, neg16, zero16))
    bv1, bi1 = plsc.sort_key_val(av1, ai1, descending=False)
    av, ai = _merge_desc(av0, ai0, bv1, bi1)

    mv_v[...] = av
    mi_v[...] = ai
    pltpu.sync_copy(mv_v, pv_hbm.at[pl.ds(w * LANES, LANES)])
    pltpu.sync_copy(mi_v, pi_hbm.at[pl.ds(w * LANES, LANES)])
    plsc.subcore_barrier()

    @pl.when(s == 0)
    def _():
        pltpu.sync_copy(pv_hbm.at[pl.ds(c * N_SUB * LANES, N_SUB * LANES)],
                        tv_v)
        pltpu.sync_copy(pi_hbm.at[pl.ds(c * N_SUB * LANES, N_SUB * LANES)],
                        ti_v)

        def merge_tile(t, carry):
            fv, fi = carry
            bv, bi = plsc.sort_key_val(tv_v[pl.ds(t * LANES, LANES)],
                                       ti_v[pl.ds(t * LANES, LANES)],
                                       descending=False)
            return _merge_desc(fv, fi, bv, bi)

        fv, fi = lax.fori_loop(1, N_SUB, merge_tile, (av, ai))
        mv_v[...] = fv
        mi_v[...] = fi
        pltpu.async_copy(mem_hbm.at[mi_v], rows_v, sem).wait()
        pltpu.sync_copy(mv_v, vals_out.at[c])
        pltpu.sync_copy(rows_v, rows_out.at[c])


@functools.cache
def _make_topk_sc():
    cp = pltpu.CompilerParams()
    if "needs_layout_passes" in pltpu.CompilerParams.__dataclass_fields__:
        cp = dataclasses.replace(cp, needs_layout_passes=False)
    return functools.partial(
        pl.kernel,
        compiler_params=cp,
        mesh=plsc.VectorSubcoreMesh(core_axis_name="c", subcore_axis_name="s"),
        out_type=[
            jax.ShapeDtypeStruct((N_CORES, LANES), jnp.float32),
            jax.ShapeDtypeStruct((N_CORES, LANES, D), jnp.float32),
            jax.ShapeDtypeStruct((N_TILES * LANES,), jnp.float32),
            jax.ShapeDtypeStruct((N_TILES * LANES,), jnp.int32),
        ],
        scratch_types=[
            pltpu.VMEM((CHUNK,), jnp.float32),
            pltpu.VMEM((CHUNK,), jnp.float32),
            pltpu.VMEM((LANES,), jnp.float32),
            pltpu.VMEM((LANES,), jnp.int32),
            pltpu.VMEM((N_SUB * LANES,), jnp.float32),
            pltpu.VMEM((N_SUB * LANES,), jnp.int32),
            pltpu.VMEM((LANES, D), jnp.float32),
            pltpu.SemaphoreType.DMA,
        ],
    )(_topk_body)


def _rnn_body(q_ref, vr_ref, vc_ref, rows_ref, wih_ref, bih_ref,
              whh_ref, bhh_ref, wg_ref, bg_ref, out_ref):
    q = q_ref[...]
    vr = vr_ref[...]
    vc = vc_ref[...]
    gt = (vc > vr).astype(jnp.int32)
    rank = jnp.sum(gt, axis=0, keepdims=True)
    sel = (lax.broadcasted_iota(jnp.int32, (K, N_TILES), 0)
           == rank).astype(jnp.float32)
    retr = lax.dot_general(
        sel, rows_ref[...], (((1,), (0,)), ((), ())),
        preferred_element_type=jnp.float32, precision=_PREC)

    x = jnp.concatenate([q, retr], axis=0)
    xp = lax.dot_general(
        x, wih_ref[...], (((1,), (0,)), ((), ())),
        preferred_element_type=jnp.float32, precision=_PREC) + bih_ref[...]

    whh = whh_ref[...]
    bhh = bhh_ref[...]
    h = jnp.zeros((1, HID), jnp.float32)
    for t in range(K + 1):
        h = jnp.tanh(xp[t:t + 1, :] + lax.dot_general(
            h, whh, (((1,), (0,)), ((), ())),
            preferred_element_type=jnp.float32, precision=_PREC) + bhh)

    gate = jax.nn.sigmoid(lax.dot_general(
        q, wg_ref[...], (((1,), (0,)), ((), ())),
        preferred_element_type=jnp.float32, precision=_PREC) + bg_ref[...])
    out_ref[...] = gate * h + (1.0 - gate) * xp[0:1, :]


def _contextualize(query_vector, vals, rows, W_ih, b_ih, W_hh, b_hh,
                   W_gate, b_gate):
    full = lambda s: pl.BlockSpec(s, lambda: tuple(0 for _ in s))
    return pl.pallas_call(
        _rnn_body,
        in_specs=[
            full((1, D)), full((1, N_TILES)), full((N_TILES, 1)),
            full((N_TILES, D)), full((D, HID)), full((1, HID)),
            full((HID, HID)), full((1, HID)), full((D, HID)), full((1, HID)),
        ],
        out_specs=full((1, HID)),
        out_shape=jax.ShapeDtypeStruct((1, HID), jnp.float32),
    )(query_vector, vals.reshape(1, N_TILES), vals.reshape(N_TILES, 1),
      rows.reshape(N_TILES, D), W_ih, b_ih.reshape(1, HID), W_hh,
      b_hh.reshape(1, HID), W_gate, b_gate.reshape(1, HID))


def kernel(query_vector, memory_vectors, coords, spatial_weights,
           W_ih, b_ih, W_hh, b_hh, W_gate, b_gate):
    sim, act = _total_sim(query_vector, spatial_weights, memory_vectors,
                          coords)
    vals, rows, _, _ = _make_topk_sc()(sim.reshape(PAD_M), act.reshape(PAD_M),
                                       memory_vectors)
    return _contextualize(query_vector, vals, rows, W_ih, b_ih, W_hh, b_hh,
                          W_gate, b_gate)
